# Initial kernel scaffold; baseline (speedup 1.0000x reference)
import jax
import jax.numpy as jnp
from jax.experimental import pallas as pl


def kernel(cloth_pos, obstacle_pos, obstacle_vertex_type):
    raise NotImplementedError("write your pallas kernel here")


# speedup vs baseline: 3.3732x; 3.3732x over previous
import functools

import jax
import jax.numpy as jnp
from jax.experimental import pallas as pl

_RADIUS = 0.1
_K = 16
_OMIT = 5
_BQ = 200
_KP = 5120


def _knn_body(cloth_ref, obt_ref, vt_ref, dists_ref, idx_ref, valid_ref,
              active_ref):
    q = cloth_ref[...]
    obt = obt_ref[...]
    vt = vt_ref[...]

    qk = jnp.dot(q.astype(jnp.bfloat16), obt.astype(jnp.bfloat16),
                 preferred_element_type=jnp.float32)
    q2 = q[:, 0:1] ** 2 + q[:, 1:2] ** 2 + q[:, 2:3] ** 2
    k2 = obt[0:1, :] ** 2 + obt[1:2, :] ** 2 + obt[2:3, :] ** 2
    d2 = jnp.maximum((q2 + k2) - 2.0 * qk, 0.0)

    lane = jax.lax.broadcasted_iota(jnp.int32, (_BQ, _KP), 1)
    bad = (vt == _OMIT).astype(jnp.int32)
    code = 2 * lane + bad
    big_i = jnp.int32(2 ** 30)
    inf_f = jnp.float32(jnp.inf)

    w = d2
    dist_cols = []
    idx_cols = []
    valid_cols = []
    for _ in range(_K):
        m = jnp.min(w, axis=1, keepdims=True)
        sel = jnp.min(jnp.where(w == m, code, big_i), axis=1,
                      keepdims=True)
        one_hot = code == sel
        w = jnp.where(one_hot, inf_f, w)
        dist = jnp.sqrt(m + 1e-12)
        dist_cols.append(dist)
        idx_cols.append(jax.lax.shift_right_logical(sel, 1))
        valid_cols.append(((dist <= _RADIUS) &
                           ((sel & 1) == 0)).astype(jnp.int32))

    dists_ref[...] = jnp.concatenate(dist_cols, axis=1)
    idx_ref[...] = jnp.concatenate(idx_cols, axis=1)
    valid_ref[...] = jnp.concatenate(valid_cols, axis=1)

    contrib_pos = (jnp.isinf(w) & (jnp.sqrt(d2 + 1e-12) <= _RADIUS) &
                   (vt != _OMIT))
    contrib = jnp.max(contrib_pos.astype(jnp.int32), axis=0,
                      keepdims=True)

    @pl.when(pl.program_id(0) == 0)
    def _init():
        active_ref[...] = contrib

    @pl.when(pl.program_id(0) > 0)
    def _acc():
        active_ref[...] = jnp.maximum(active_ref[...], contrib)


@functools.partial(jax.jit, static_argnames=())
def kernel(cloth_pos, obstacle_pos, obstacle_vertex_type):
    q_n = cloth_pos.shape[0]
    k_n = obstacle_pos.shape[0]
    pad = _KP - k_n
    obt = jnp.concatenate(
        [obstacle_pos, jnp.full((pad, 3), 1e4, jnp.float32)], axis=0).T
    vt = jnp.concatenate(
        [obstacle_vertex_type, jnp.full((pad,), _OMIT, jnp.int32)]
    ).reshape(1, _KP)

    grid = q_n // _BQ
    dists, idx, valid_i, active = pl.pallas_call(
        _knn_body,
        grid=(grid,),
        in_specs=[
            pl.BlockSpec((_BQ, 3), lambda i: (i, 0)),
            pl.BlockSpec((3, _KP), lambda i: (0, 0)),
            pl.BlockSpec((1, _KP), lambda i: (0, 0)),
        ],
        out_specs=[
            pl.BlockSpec((_BQ, _K), lambda i: (i, 0)),
            pl.BlockSpec((_BQ, _K), lambda i: (i, 0)),
            pl.BlockSpec((_BQ, _K), lambda i: (i, 0)),
            pl.BlockSpec((1, _KP), lambda i: (0, 0)),
        ],
        out_shape=[
            jax.ShapeDtypeStruct((q_n, _K), jnp.float32),
            jax.ShapeDtypeStruct((q_n, _K), jnp.int32),
            jax.ShapeDtypeStruct((q_n, _K), jnp.int32),
            jax.ShapeDtypeStruct((1, _KP), jnp.int32),
        ],
    )(cloth_pos, obt, vt)

    indices_from = jnp.broadcast_to(
        jnp.arange(q_n, dtype=jnp.int32)[:, None], idx.shape)
    edges_direct = jnp.stack([indices_from, idx], axis=0)
    edges_inverse = jnp.stack([idx, indices_from], axis=0)
    valid = valid_i.astype(jnp.bool_)
    obstacle_active_mask = active[0, :k_n] > 0
    return dists, edges_direct, edges_inverse, valid, obstacle_active_mask
